# baseline (device time: 28658 ns/iter reference)
import jax
import jax.numpy as jnp
from jax import lax
from jax.experimental import pallas as pl
from jax.experimental.pallas import tpu as pltpu

N_DEV = 4
Dh = 64
GQA_GROUP = 4


def kernel(x, Wq, Wo, Wk, Wv):
    B, Sq, D = x.shape
    BSq = B * Sq
    dq = Wq.shape[1]
    Hq_loc = dq // Dh
    kv_cols = (Hq_loc // GQA_GROUP) * Dh
    HALF = BSq // 2
    QTR = BSq // 4
    CH = D // 2

    def body(x_ref, wq_ref, wo_ref, wk_ref, wv_ref, out_ref,
             qv_ref, kv_ref, vv_ref, attn_ref, acc_ref,
             send_ref, recv_ref, send_sems, recv_sems):
        p = lax.axis_index("i")
        pa = jnp.bitwise_xor(p, 1)
        pb = 3 - p

        kbP = jnp.where((p == 1) | (p == 2), 1, 0)
        kqP = jnp.where(p >= 2, 1, 0)
        kbQ = jnp.where(p >= 2, 1, 0)
        kqQ = lax.rem(p, 2)
        partsP = (pa, pb, pb, pa)
        partsQ = (pb, pa, pa, pb)
        fb = 1 - kbP

        xf = x_ref[:].reshape(BSq, D).astype(jnp.bfloat16)
        kv0 = p * kv_cols
        qv_ref[:] = jnp.dot(xf, wq_ref[:].astype(jnp.bfloat16),
                            preferred_element_type=jnp.float32
                            ).astype(jnp.bfloat16)
        kv_ref[:] = jnp.dot(xf, wk_ref[:, pl.ds(kv0, kv_cols)]
                            .astype(jnp.bfloat16),
                            preferred_element_type=jnp.float32
                            ).astype(jnp.bfloat16)
        vv_ref[:] = jnp.dot(xf, wv_ref[:, pl.ds(kv0, kv_cols)]
                            .astype(jnp.bfloat16),
                            preferred_element_type=jnp.float32
                            ).astype(jnp.bfloat16)
        wob = wo_ref[:].astype(jnp.bfloat16)

        barrier_sem = pltpu.get_barrier_semaphore()
        for nbr in (pa, pb):
            pl.semaphore_signal(
                barrier_sem, inc=1,
                device_id=(nbr,), device_id_type=pl.DeviceIdType.MESH,
            )
        pl.semaphore_wait(barrier_sem, 2)

        def step1_rdma(bi, kb, partner):
            return pltpu.make_async_remote_copy(
                src_ref=send_ref.at[0, bi, pl.ds(0, HALF)],
                dst_ref=recv_ref.at[0, bi, pl.ds(0, HALF)],
                send_sem=send_sems.at[0, bi],
                recv_sem=recv_sems.at[0, bi],
                device_id=(partner,),
                device_id_type=pl.DeviceIdType.MESH,
            )

        def launch_step1(bi, kb, col0, partner):
            send_ref[0, bi, pl.ds(0, HALF)] = (
                acc_ref[pl.ds((1 - kb) * HALF, HALF), pl.ds(col0, CH)]
                .astype(jnp.bfloat16))
            step1_rdma(bi, kb, partner).start()

        for bb in range(B):
            brow = (fb if bb == 0 else 1 - fb) * HALF
            for g in range(Hq_loc // GQA_GROUP):
                kc = g * Dh
                k = kv_ref[pl.ds(brow, Sq), kc:kc + Dh]
                v = vv_ref[pl.ds(brow, Sq), kc:kc + Dh]
                qs = jnp.concatenate(
                    [qv_ref[pl.ds(brow, Sq),
                            pl.ds((g * GQA_GROUP + j) * Dh, Dh)]
                     for j in range(GQA_GROUP)], axis=0)
                s = lax.dot_general(
                    qs, k, (((1,), (1,)), ((), ())),
                    preferred_element_type=jnp.float32,
                )
                pj = jnp.exp(s * 0.125)
                l = jnp.sum(pj, axis=1, keepdims=True)
                o = jnp.dot(pj.astype(jnp.bfloat16), v,
                            preferred_element_type=jnp.float32) / l
                ob = o.astype(jnp.bfloat16)
                for j in range(GQA_GROUP):
                    attn_ref[pl.ds(brow, Sq),
                             pl.ds((g * GQA_GROUP + j) * Dh, Dh)] = (
                        ob[j * Sq:(j + 1) * Sq, :])
            acc_ref[pl.ds(brow, HALF), :] = jnp.dot(
                attn_ref[pl.ds(brow, HALF), :], wob,
                preferred_element_type=jnp.float32)

            if bb == 0:
                launch_step1(0, kbP, 0, partsP[0])

                @pl.when(kbQ == kbP)
                def _():
                    launch_step1(1, kbQ, CH, partsQ[0])
            else:
                @pl.when(kbQ != kbP)
                def _():
                    launch_step1(1, kbQ, CH, partsQ[0])

        for bi, (kb, col0, partner) in enumerate(
                ((kbP, 0, partsP[0]), (kbQ, CH, partsQ[0]))):
            rdma = step1_rdma(bi, kb, partner)
            rdma.wait_recv()
            got = recv_ref[0, bi, pl.ds(0, HALF)].astype(jnp.float32)
            dst = (pl.ds(kb * HALF, HALF), pl.ds(col0, CH))
            acc_ref[dst] = acc_ref[dst] + got
            rdma.wait_send()

        plan = [
            (QTR,  lambda kb, kq: kb * HALF + (1 - kq) * QTR,
                   lambda kb, kq: kb * HALF + kq * QTR, True),
            (QTR,  lambda kb, kq: kb * HALF + kq * QTR,
                   lambda kb, kq: kb * HALF + (1 - kq) * QTR, False),
            (HALF, lambda kb, kq: kb * HALF,
                   lambda kb, kq: (1 - kb) * HALF, False),
        ]

        for s0, (n, src_row, apply_row, is_add) in enumerate(plan):
            s = s0 + 1
            rdmas = []
            cfgs = ((kbP, kqP, 0, partsP[s]), (kbQ, kqQ, CH, partsQ[s]))
            for bi, (kb, kq, col0, partner) in enumerate(cfgs):
                send_ref[s, bi, pl.ds(0, n)] = (
                    acc_ref[pl.ds(src_row(kb, kq), n), pl.ds(col0, CH)]
                    .astype(jnp.bfloat16))
                rdma = pltpu.make_async_remote_copy(
                    src_ref=send_ref.at[s, bi, pl.ds(0, n)],
                    dst_ref=recv_ref.at[s, bi, pl.ds(0, n)],
                    send_sem=send_sems.at[s, bi],
                    recv_sem=recv_sems.at[s, bi],
                    device_id=(partner,),
                    device_id_type=pl.DeviceIdType.MESH,
                )
                rdma.start()
                rdmas.append(rdma)
            for bi, (kb, kq, col0, partner) in enumerate(cfgs):
                rdmas[bi].wait_recv()
                got = recv_ref[s, bi, pl.ds(0, n)].astype(jnp.float32)
                dst = (pl.ds(apply_row(kb, kq), n), pl.ds(col0, CH))
                if is_add:
                    acc_ref[dst] = acc_ref[dst] + got
                else:
                    acc_ref[dst] = got
            for bi in range(2):
                rdmas[bi].wait_send()

        out_ref[:] = acc_ref[:].reshape(B, Sq, D)

    return pl.pallas_call(
        body,
        out_shape=jax.ShapeDtypeStruct((B, Sq, D), jnp.float32),
        in_specs=[pl.BlockSpec(memory_space=pltpu.VMEM)] * 5,
        out_specs=pl.BlockSpec(memory_space=pltpu.VMEM),
        scratch_shapes=[
            pltpu.VMEM((BSq, dq), jnp.bfloat16),
            pltpu.VMEM((BSq, kv_cols), jnp.bfloat16),
            pltpu.VMEM((BSq, kv_cols), jnp.bfloat16),
            pltpu.VMEM((BSq, dq), jnp.bfloat16),
            pltpu.VMEM((BSq, D), jnp.float32),
            pltpu.VMEM((4, 2, HALF, CH), jnp.bfloat16),
            pltpu.VMEM((4, 2, HALF, CH), jnp.bfloat16),
            pltpu.SemaphoreType.DMA((4, 2)),
            pltpu.SemaphoreType.DMA((4, 2)),
        ],
        compiler_params=pltpu.CompilerParams(collective_id=0),
    )(x, Wq, Wo, Wk, Wv)
